# baseline (speedup 1.0000x reference)
import functools

import numpy as _np

import jax
import jax.numpy as jnp
from jax import lax
from jax.experimental import pallas as pl
from jax.experimental.pallas import tpu as pltpu
from jax.experimental.pallas import tpu_sc as plsc


def kernel(atom, edge_index, W, b):
    f32 = jnp.float32
    N, D_IN = atom.shape
    D_OUT = W.shape[0]
    E = edge_index.shape[1]

    NC, NS = 2, 16
    NW = NC * NS
    CHF = 512
    NCH = -(-E // (NW * CHF))
    if NCH % 6:
        NCH += 6 - NCH % 6
    T = NCH * CHF
    E_pad = NW * T

    NT = 128 * (-(-(N + 1) // (NS * 128)))
    NP = NS * NT
    ZR = 128

    NCHD = 16
    CHD = T // NCHD
    assert T % NCHD == 0 and CHD % 16 == 0
    assert NT % ZR == 0

    pad = E_pad - E
    seq = jnp.arange(pad, dtype=jnp.int32)
    dummy_row = seq % N
    dummy_col = N + seq % (NP - N)
    row1 = jnp.concatenate([edge_index[0], dummy_row])
    col1 = jnp.concatenate([edge_index[1], dummy_col])

    mesh = plsc.VectorSubcoreMesh(core_axis_name="c", subcore_axis_name="s",
                                  num_cores=NC, num_subcores=NS)
    sc_params = pltpu.CompilerParams(use_tc_tiling_on_sc=False)

    @functools.partial(
        pl.kernel,
        mesh=mesh,
        out_type=jax.ShapeDtypeStruct((NC * NP,), f32),
        compiler_params=sc_params,
        scratch_types=[
            pltpu.VMEM((CHD,), jnp.int32),
            pltpu.VMEM((CHD,), jnp.int32),
            pltpu.VMEM((CHD,), f32),
            pltpu.VMEM((NT,), f32),
            pltpu.VMEM_SHARED((NP,), f32),
            pltpu.SemaphoreType.DMA,
            pltpu.SemaphoreType.DMA,
        ],
    )
    def deg_kernel(col_hbm, deg_hbm, colva, colvb, onesv, zv, degs,
                   sema, semb):
        c = lax.axis_index("c")
        s = lax.axis_index("s")
        wid = c * NS + s
        ones16 = jnp.full((16,), 1.0, f32)
        zero16 = jnp.zeros((16,), f32)

        def fill_ones(q, carry):
            onesv[pl.ds(q * 16, 16)] = ones16
            return carry

        lax.fori_loop(0, CHD // 16, fill_ones, 0)

        def fill_zero(q, carry):
            zv[pl.ds(q * 16, 16)] = zero16
            return carry

        lax.fori_loop(0, NT // 16, fill_zero, 0)
        pltpu.sync_copy(zv, degs.at[pl.ds(s * NT, NT)])
        plsc.subcore_barrier()

        base = wid * T

        def ld(ci, buf, sem):
            return pltpu.async_copy(
                col_hbm.at[pl.ds(base + ci * CHD, CHD)], buf, sem)

        ld(0, colva, sema)

        def chunk2(i, carry):
            c0 = 2 * i
            pltpu.make_async_copy(col_hbm, colva, sema).wait()

            @pl.when(c0 + 1 < NCHD)
            def _():
                ld(c0 + 1, colvb, semb)

            pltpu.sync_copy(onesv, degs.at[colva], add=True)

            @pl.when(c0 + 2 < NCHD)
            def _():
                ld(c0 + 2, colva, sema)

            @pl.when(c0 + 1 < NCHD)
            def _():
                pltpu.make_async_copy(col_hbm, colvb, semb).wait()
                pltpu.sync_copy(onesv, degs.at[colvb], add=True)

            return carry

        lax.fori_loop(0, NCHD // 2, chunk2, 0)
        plsc.subcore_barrier()
        pltpu.sync_copy(degs.at[pl.ds(s * NT, NT)], zv)
        pltpu.sync_copy(zv, deg_hbm.at[pl.ds(c * NP + s * NT, NT)])

    degp = deg_kernel(col1).reshape(NC * NP // 128, 128)

    GN = NP // 2048
    BR = 2048
    DR = BR // 128

    def lin_body(atom_ref, wt_ref, b_ref, dg0_ref, dg1_ref, y_ref, dinv_ref):
        x = jnp.dot(atom_ref[...], wt_ref[...], preferred_element_type=f32)
        x = x + b_ref[...]
        deg = 1.0 + dg0_ref[...] + dg1_ref[...]
        dinv = lax.rsqrt(deg)
        dinv_ref[...] = dinv
        dinv_t = lax.transpose(dinv, (1, 0))
        pieces = []
        for a in range(DR):
            xa = lax.slice(x, (128 * a, 0), (128 * (a + 1), D_OUT))
            da = lax.slice(dinv_t, (0, a), (128, a + 1))
            pieces.append(xa * da)
        yv = lax.concatenate(pieces, 0)
        y3 = yv.reshape(BR // 8, 8, D_OUT)
        packed = lax.concatenate(
            [lax.squeeze(lax.slice(y3, (0, u, 0), (BR // 8, u + 1, D_OUT)),
                         (1,)) for u in range(8)], 1)
        y_ref[...] = packed

    yp, dinvp = pl.pallas_call(
        lin_body,
        grid=(GN,),
        in_specs=[
            pl.BlockSpec((BR, D_IN), lambda i: (i, 0)),
            pl.BlockSpec((D_IN, D_OUT), lambda i: (0, 0)),
            pl.BlockSpec((1, D_OUT), lambda i: (0, 0)),
            pl.BlockSpec((DR, 128), lambda i: (i, 0)),
            pl.BlockSpec((DR, 128), lambda i: (GN + i, 0)),
        ],
        out_specs=[
            pl.BlockSpec((BR // 8, 128), lambda i: (i, 0)),
            pl.BlockSpec((DR, 128), lambda i: (i, 0)),
        ],
        out_shape=[
            jax.ShapeDtypeStruct((NP // 8, 128), f32),
            jax.ShapeDtypeStruct((NP // 128, 128), f32),
        ],
    )(atom, W.T, b.reshape(1, D_OUT), degp, degp)
    y = yp.reshape(NP, D_OUT)

    @functools.partial(
        pl.kernel,
        mesh=mesh,
        out_type=jax.ShapeDtypeStruct((NC * NP, D_OUT), f32),
        compiler_params=sc_params,
        scratch_types=[
            [pltpu.VMEM((CHF,), jnp.int32) for _ in range(6)],
            [pltpu.VMEM((CHF,), jnp.int32) for _ in range(6)],
            [pltpu.VMEM((CHF, D_OUT), f32) for _ in range(3)],
            pltpu.VMEM_SHARED((NP, D_OUT), f32),
            [pltpu.SemaphoreType.DMA for _ in range(6)],
            [pltpu.SemaphoreType.DMA for _ in range(3)],
            [pltpu.SemaphoreType.DMA for _ in range(3)],
        ],
    )
    def scat_kernel(y_hbm, row_hbm, col_hbm, acc_hbm,
                    rows, cols, msgs, accs, isems, gsem, ssem):
        c = lax.axis_index("c")
        s = lax.axis_index("s")
        wid = c * NS + s
        zero16 = jnp.zeros((D_OUT,), f32)
        WBR = NT // 14

        def fz(q, carry):
            msgs[0][q, :] = zero16
            return carry

        lax.fori_loop(0, WBR, fz, 0)

        def zc(k, carry):
            pltpu.sync_copy(msgs[0].at[pl.ds(0, WBR)],
                            accs.at[pl.ds(s * NT + k * WBR, WBR)])
            return carry

        lax.fori_loop(0, NT // WBR, zc, 0)
        plsc.subcore_barrier()

        base = wid * T

        def start_idx(gi, x):
            pltpu.async_copy(row_hbm.at[pl.ds(base + gi * CHF, CHF)],
                             rows[x], isems[x])
            pltpu.async_copy(col_hbm.at[pl.ds(base + gi * CHF, CHF)],
                             cols[x], isems[x])

        def wait_idx(x):
            pltpu.make_async_copy(row_hbm, rows[x], isems[x]).wait()
            pltpu.make_async_copy(row_hbm, cols[x], isems[x]).wait()

        def start_gather(x, q):
            pltpu.async_copy(y_hbm.at[rows[x]], msgs[q], gsem[q])

        def wait_gather(q):
            pltpu.make_async_copy(y_hbm, msgs[q], gsem[q]).wait()

        def start_scatter(x, q):
            pltpu.async_copy(msgs[q], accs.at[cols[x]], ssem[q], add=True)

        def wait_scatter(x, q):
            pltpu.make_async_copy(msgs[q], accs.at[cols[x]], ssem[q]).wait()

        for g in range(6):
            start_idx(g, g)
        wait_idx(0)
        start_gather(0, 0)

        def hexa(j, carry):
            for k in range(6):
                q = k % 3
                nq = (k + 1) % 3
                xf = (k + 4) % 6

                if k < 2:
                    @pl.when(j > 0)
                    def _():
                        wait_scatter(xf, nq)
                        start_idx(6 * j + k + 4, xf)
                else:
                    wait_scatter(xf, nq)

                    @pl.when(6 * j + k + 4 < NCH)
                    def _():
                        start_idx(6 * j + k + 4, xf)

                if k == 5:
                    @pl.when(j + 1 < NCH // 6)
                    def _():
                        wait_idx(0)
                        start_gather(0, nq)
                else:
                    wait_idx(k + 1)
                    start_gather(k + 1, nq)

                wait_gather(q)
                start_scatter(k, q)
            return carry

        lax.fori_loop(0, NCH // 6, hexa, 0)
        wait_scatter(4, 1)
        wait_scatter(5, 2)
        plsc.subcore_barrier()

        def wb(k, carry):
            pltpu.sync_copy(accs.at[pl.ds(s * NT + k * WBR, WBR)],
                            msgs[0].at[pl.ds(0, WBR)])
            pltpu.sync_copy(msgs[0].at[pl.ds(0, WBR)],
                            acc_hbm.at[pl.ds(c * NP + s * NT + k * WBR, WBR)])
            return carry

        lax.fori_loop(0, NT // WBR, wb, 0)

    acc = scat_kernel(y, row1, col1)
    accp = acc.reshape(NC * NP // 8, 128)

    PB = BR // 8
    NPB = NP // 8 // PB

    sel_np = _np.zeros((8 * 128, D_OUT), _np.float32)
    for u in range(8):
        for j in range(D_OUT):
            sel_np[u * 128 + D_OUT * u + j, j] = 1.0
    sel = jnp.asarray(sel_np)

    def out_body(a0_ref, a1_ref, y_ref, dg0_ref, dg1_ref, sel_ref, o_ref):
        tp = a0_ref[...] + a1_ref[...] + y_ref[...]
        rep = jnp.broadcast_to(tp[:, None, :], (PB, 8, 128)).reshape(BR, 128)
        sub = lax.broadcasted_iota(jnp.int32, (BR, 1), 0) % 8
        sel_all = sel_ref[...]
        t = jnp.zeros((BR, D_OUT), f32)
        for u in range(8):
            su = lax.slice(sel_all, (u * 128, 0), (u * 128 + 128, D_OUT))
            pu = jnp.dot(rep, su, preferred_element_type=f32,
                         precision=lax.Precision.HIGHEST)
            t = jnp.where(sub == u, pu, t)
        deg = 1.0 + dg0_ref[...] + dg1_ref[...]
        dinv_t = lax.transpose(lax.rsqrt(deg), (1, 0))
        for a in range(DR):
            ta = lax.slice(t, (128 * a, 0), (128 * (a + 1), D_OUT))
            da = lax.slice(dinv_t, (0, a), (128, a + 1))
            o_ref[pl.ds(128 * a, 128), :] = jnp.maximum(ta * da, 0.0)

    out = pl.pallas_call(
        out_body,
        grid=(GN,),
        in_specs=[
            pl.BlockSpec((PB, 128), lambda i: (i, 0)),
            pl.BlockSpec((PB, 128), lambda i: (NPB + i, 0)),
            pl.BlockSpec((PB, 128), lambda i: (i, 0)),
            pl.BlockSpec((DR, 128), lambda i: (i, 0)),
            pl.BlockSpec((DR, 128), lambda i: (GN + i, 0)),
            pl.BlockSpec((8 * 128, D_OUT), lambda i: (0, 0)),
        ],
        out_specs=pl.BlockSpec((BR, D_OUT), lambda i: (i, 0)),
        out_shape=jax.ShapeDtypeStruct((N, D_OUT), f32),
    )(accp, accp, yp, degp, degp, sel)

    return out


# speedup vs baseline: 1.6697x; 1.6697x over previous
import functools

import numpy as _np

import jax
import jax.numpy as jnp
from jax import lax
from jax.experimental import pallas as pl
from jax.experimental.pallas import tpu as pltpu
from jax.experimental.pallas import tpu_sc as plsc


def kernel(atom, edge_index, W, b):
    f32 = jnp.float32
    N, D_IN = atom.shape
    D_OUT = W.shape[0]
    E = edge_index.shape[1]

    NC, NS = 2, 16
    NW = NC * NS
    CHF = 512
    NCH = -(-E // (NW * CHF))
    if NCH % 6:
        NCH += 6 - NCH % 6
    T = NCH * CHF
    E_pad = NW * T

    NT = 128 * (-(-(N + 1) // (NS * 128)))
    NP = NS * NT
    ZR = 128

    NCHD = 16
    CHD = T // NCHD
    assert T % NCHD == 0 and CHD % 16 == 0
    assert NT % ZR == 0

    pad = E_pad - E
    seq = jnp.arange(pad, dtype=jnp.int32)
    dummy_row = seq % N
    dummy_col = N + seq % (NP - N)
    row1 = jnp.concatenate([edge_index[0], dummy_row])
    col1 = jnp.concatenate([edge_index[1], dummy_col])

    mesh = plsc.VectorSubcoreMesh(core_axis_name="c", subcore_axis_name="s",
                                  num_cores=NC, num_subcores=NS)
    sc_params = pltpu.CompilerParams(use_tc_tiling_on_sc=False)

    @functools.partial(
        pl.kernel,
        mesh=mesh,
        out_type=jax.ShapeDtypeStruct((NC * NP,), f32),
        compiler_params=sc_params,
        scratch_types=[
            pltpu.VMEM((CHD,), jnp.int32),
            pltpu.VMEM((CHD,), jnp.int32),
            pltpu.VMEM((CHD,), f32),
            pltpu.VMEM((NT,), f32),
            pltpu.VMEM_SHARED((NP,), f32),
            pltpu.SemaphoreType.DMA,
            pltpu.SemaphoreType.DMA,
        ],
    )
    def deg_kernel(col_hbm, deg_hbm, colva, colvb, onesv, zv, degs,
                   sema, semb):
        c = lax.axis_index("c")
        s = lax.axis_index("s")
        wid = c * NS + s
        ones16 = jnp.full((16,), 1.0, f32)
        zero16 = jnp.zeros((16,), f32)

        def fill_ones(q, carry):
            onesv[pl.ds(q * 16, 16)] = ones16
            return carry

        lax.fori_loop(0, CHD // 16, fill_ones, 0)

        def fill_zero(q, carry):
            zv[pl.ds(q * 16, 16)] = zero16
            return carry

        lax.fori_loop(0, NT // 16, fill_zero, 0)
        pltpu.sync_copy(zv, degs.at[pl.ds(s * NT, NT)])
        plsc.subcore_barrier()

        base = wid * T

        def ld(ci, buf, sem):
            return pltpu.async_copy(
                col_hbm.at[pl.ds(base + ci * CHD, CHD)], buf, sem)

        ld(0, colva, sema)

        def chunk2(i, carry):
            c0 = 2 * i
            pltpu.make_async_copy(col_hbm, colva, sema).wait()

            @pl.when(c0 + 1 < NCHD)
            def _():
                ld(c0 + 1, colvb, semb)

            pltpu.sync_copy(onesv, degs.at[colva], add=True)

            @pl.when(c0 + 2 < NCHD)
            def _():
                ld(c0 + 2, colva, sema)

            @pl.when(c0 + 1 < NCHD)
            def _():
                pltpu.make_async_copy(col_hbm, colvb, semb).wait()
                pltpu.sync_copy(onesv, degs.at[colvb], add=True)

            return carry

        lax.fori_loop(0, NCHD // 2, chunk2, 0)
        plsc.subcore_barrier()
        pltpu.sync_copy(degs.at[pl.ds(s * NT, NT)], zv)
        pltpu.sync_copy(zv, deg_hbm.at[pl.ds(c * NP + s * NT, NT)])

    degp = deg_kernel(col1).reshape(NC * NP // 128, 128)

    GN = NP // 2048
    BR = 2048
    DR = BR // 128

    def lin_body(atom_ref, wt_ref, b_ref, dg0_ref, dg1_ref, y_ref, dinv_ref):
        x = jnp.dot(atom_ref[...], wt_ref[...], preferred_element_type=f32)
        x = x + b_ref[...]
        deg = 1.0 + dg0_ref[...] + dg1_ref[...]
        dinv = lax.rsqrt(deg)
        dinv_ref[...] = dinv
        dinv_t = lax.transpose(dinv, (1, 0))
        pieces = []
        for a in range(DR):
            xa = lax.slice(x, (128 * a, 0), (128 * (a + 1), D_OUT))
            da = lax.slice(dinv_t, (0, a), (128, a + 1))
            pieces.append(xa * da)
        yv = lax.concatenate(pieces, 0)
        y3 = yv.reshape(BR // 8, 8, D_OUT)
        packed = lax.concatenate(
            [lax.squeeze(lax.slice(y3, (0, u, 0), (BR // 8, u + 1, D_OUT)),
                         (1,)) for u in range(8)], 1)
        y_ref[...] = packed

    yp, dinvp = pl.pallas_call(
        lin_body,
        grid=(GN,),
        in_specs=[
            pl.BlockSpec((BR, D_IN), lambda i: (i, 0)),
            pl.BlockSpec((D_IN, D_OUT), lambda i: (0, 0)),
            pl.BlockSpec((1, D_OUT), lambda i: (0, 0)),
            pl.BlockSpec((DR, 128), lambda i: (i, 0)),
            pl.BlockSpec((DR, 128), lambda i: (GN + i, 0)),
        ],
        out_specs=[
            pl.BlockSpec((BR // 8, 128), lambda i: (i, 0)),
            pl.BlockSpec((DR, 128), lambda i: (i, 0)),
        ],
        out_shape=[
            jax.ShapeDtypeStruct((NP // 8, 128), f32),
            jax.ShapeDtypeStruct((NP // 128, 128), f32),
        ],
    )(atom, W.T, b.reshape(1, D_OUT), degp, degp)
    y = yp.reshape(NP, D_OUT)

    @functools.partial(
        pl.kernel,
        mesh=mesh,
        out_type=jax.ShapeDtypeStruct((NC * NP, D_OUT), f32),
        compiler_params=sc_params,
        scratch_types=[
            [pltpu.VMEM((CHF,), jnp.int32) for _ in range(6)],
            [pltpu.VMEM((CHF,), jnp.int32) for _ in range(6)],
            [pltpu.VMEM((CHF, D_OUT), f32) for _ in range(3)],
            pltpu.VMEM_SHARED((NP, D_OUT), f32),
            [pltpu.SemaphoreType.DMA for _ in range(6)],
            [pltpu.SemaphoreType.DMA for _ in range(3)],
            [pltpu.SemaphoreType.DMA for _ in range(3)],
        ],
    )
    def scat_kernel(y_hbm, row_hbm, col_hbm, acc_hbm,
                    rows, cols, msgs, accs, isems, gsem, ssem):
        c = lax.axis_index("c")
        s = lax.axis_index("s")
        wid = c * NS + s
        zero16 = jnp.zeros((D_OUT,), f32)
        WBR = NT // 14

        def fz(q, carry):
            msgs[0][q, :] = zero16
            return carry

        lax.fori_loop(0, WBR, fz, 0)

        def zc(k, carry):
            pltpu.sync_copy(msgs[0].at[pl.ds(0, WBR)],
                            accs.at[pl.ds(s * NT + k * WBR, WBR)])
            return carry

        lax.fori_loop(0, NT // WBR, zc, 0)
        plsc.subcore_barrier()

        base = wid * T

        def start_idx(gi, x):
            pltpu.async_copy(row_hbm.at[pl.ds(base + gi * CHF, CHF)],
                             rows[x], isems[x])
            pltpu.async_copy(col_hbm.at[pl.ds(base + gi * CHF, CHF)],
                             cols[x], isems[x])

        def wait_idx(x):
            pltpu.make_async_copy(row_hbm, rows[x], isems[x]).wait()
            pltpu.make_async_copy(row_hbm, cols[x], isems[x]).wait()

        def start_gather(x, q):
            pltpu.async_copy(y_hbm.at[rows[x]], msgs[q], gsem[q])

        def wait_gather(q):
            pltpu.make_async_copy(y_hbm, msgs[q], gsem[q]).wait()

        def start_scatter(x, q):
            pltpu.async_copy(msgs[q], accs.at[cols[x]], ssem[q], add=True)

        def wait_scatter(x, q):
            pltpu.make_async_copy(msgs[q], accs.at[cols[x]], ssem[q]).wait()

        for g in range(6):
            start_idx(g, g)
        wait_idx(0)
        start_gather(0, 0)

        def hexa(j, carry):
            for k in range(6):
                q = k % 3
                nq = (k + 1) % 3
                xf = (k + 4) % 6

                if k < 2:
                    @pl.when(j > 0)
                    def _():
                        wait_scatter(xf, nq)
                        start_idx(6 * j + k + 4, xf)
                else:
                    wait_scatter(xf, nq)

                    @pl.when(6 * j + k + 4 < NCH)
                    def _():
                        start_idx(6 * j + k + 4, xf)

                if k == 5:
                    @pl.when(j + 1 < NCH // 6)
                    def _():
                        wait_idx(0)
                        start_gather(0, nq)
                else:
                    wait_idx(k + 1)
                    start_gather(k + 1, nq)

                wait_gather(q)
                start_scatter(k, q)
            return carry

        lax.fori_loop(0, NCH // 6, hexa, 0)
        wait_scatter(4, 1)
        wait_scatter(5, 2)
        plsc.subcore_barrier()

        def wb(k, carry):
            pltpu.sync_copy(accs.at[pl.ds(s * NT + k * WBR, WBR)],
                            msgs[0].at[pl.ds(0, WBR)])
            pltpu.sync_copy(msgs[0].at[pl.ds(0, WBR)],
                            acc_hbm.at[pl.ds(c * NP + s * NT + k * WBR, WBR)])
            return carry

        lax.fori_loop(0, NT // WBR, wb, 0)

    acc = scat_kernel(y, row1, col1)
    accp = acc.reshape(NC * NP // 8, 128)

    PB = BR // 8
    NPB = NP // 8 // PB

    sel_np = _np.zeros((8 * 128, D_OUT), _np.float32)
    for u in range(8):
        for j in range(D_OUT):
            sel_np[u * 128 + D_OUT * u + j, j] = 1.0
    sel = jnp.asarray(sel_np)

    def out_body(a0_ref, a1_ref, y_ref, dg0_ref, dg1_ref, sel_ref, o_ref):
        tp = a0_ref[...] + a1_ref[...] + y_ref[...]
        rep = jnp.broadcast_to(tp[:, None, :], (PB, 8, 128)).reshape(BR, 128)
        sub = lax.broadcasted_iota(jnp.int32, (BR, 1), 0) % 8
        sel_all = sel_ref[...]
        t = jnp.zeros((BR, D_OUT), f32)
        for u in range(8):
            su = lax.slice(sel_all, (u * 128, 0), (u * 128 + 128, D_OUT))
            pu = jnp.dot(rep, su, preferred_element_type=f32)
            t = jnp.where(sub == u, pu, t)
        deg = 1.0 + dg0_ref[...] + dg1_ref[...]
        dinv_t = lax.transpose(lax.rsqrt(deg), (1, 0))
        for a in range(DR):
            ta = lax.slice(t, (128 * a, 0), (128 * (a + 1), D_OUT))
            da = lax.slice(dinv_t, (0, a), (128, a + 1))
            o_ref[pl.ds(128 * a, 128), :] = jnp.maximum(ta * da, 0.0)

    out = pl.pallas_call(
        out_body,
        grid=(GN,),
        in_specs=[
            pl.BlockSpec((PB, 128), lambda i: (i, 0)),
            pl.BlockSpec((PB, 128), lambda i: (NPB + i, 0)),
            pl.BlockSpec((PB, 128), lambda i: (i, 0)),
            pl.BlockSpec((DR, 128), lambda i: (i, 0)),
            pl.BlockSpec((DR, 128), lambda i: (GN + i, 0)),
            pl.BlockSpec((8 * 128, D_OUT), lambda i: (0, 0)),
        ],
        out_specs=pl.BlockSpec((BR, D_OUT), lambda i: (i, 0)),
        out_shape=jax.ShapeDtypeStruct((N, D_OUT), f32),
    )(accp, accp, yp, degp, degp, sel)

    return out
